# baseline (speedup 1.0000x reference)
import jax
import jax.numpy as jnp
from jax import lax
from jax.experimental import pallas as pl
from jax.experimental.pallas import tpu as pltpu
from jax.experimental.pallas import tpu_sc as plsc

N = 10000
E = 160000
D = 256
HALF = 128
ROWS = 1000
NBLK = N // ROWS

NC = 2
NS = 16
CH = 64
NCH = 158
EPT = NCH * CH
E_PAD = NS * EPT
STRIPE = 632
TBL = NS * STRIPE
STRIPE_LAST = N - (NS - 1) * STRIPE


def _sc_agg(xh0, xh1, src3, dst3, zeros_h, zo16_h, with_cnt):
    mesh = plsc.VectorSubcoreMesh(core_axis_name="c", subcore_axis_name="s")

    def body(*refs):
        if with_cnt:
            (xh0_r, xh1_r, src_h, dst_h, zr_h, zo16_r, agg_out, cnt_out,
             agg_sh, src_v, dst_v, rows_v, rows_b, sem, semb,
             cnt_sh, ones_v, semc) = refs
        else:
            (xh0_r, xh1_r, src_h, dst_h, zr_h, zo16_r, agg_out,
             agg_sh, src_v, dst_v, rows_v, rows_b, sem, semb) = refs
        cid = lax.axis_index("c")
        sid = lax.axis_index("s")
        base = sid * STRIPE

        pltpu.sync_copy(zr_h, agg_sh.at[pl.ds(base, STRIPE)])
        pltpu.sync_copy(src_h.at[sid], src_v)
        pltpu.sync_copy(dst_h.at[sid], dst_v)
        if with_cnt:
            @pl.when(cid == 0)
            def _():
                pltpu.sync_copy(zo16_r.at[pl.ds(0, STRIPE)],
                                cnt_sh.at[pl.ds(base, STRIPE)])
                pltpu.sync_copy(zo16_r.at[pl.ds(STRIPE, CH)], ones_v)

        def run(x_half, count):
            pltpu.async_copy(x_half.at[src_v.at[0]], rows_v, sem)

            def scat(j, buf):
                if count:
                    ac = pltpu.async_copy(ones_v, cnt_sh.at[dst_v.at[j]],
                                          semc, add=True)
                pltpu.sync_copy(buf, agg_sh.at[dst_v.at[j]], add=True)
                if count:
                    ac.wait()

            def pair(k, carry):
                j0 = 2 * k
                g1 = pltpu.async_copy(x_half.at[src_v.at[j0 + 1]],
                                      rows_b, semb)
                pltpu.make_async_copy(x_half.at[src_v.at[j0]],
                                      rows_v, sem).wait()
                scat(j0, rows_v)

                @pl.when(j0 + 2 < NCH)
                def _():
                    pltpu.async_copy(x_half.at[src_v.at[j0 + 2]],
                                     rows_v, sem)
                g1.wait()
                scat(j0 + 1, rows_b)
                return carry
            lax.fori_loop(0, NCH // 2, pair, 0)

        plsc.subcore_barrier()

        @pl.when(cid == 0)
        def _():
            run(xh0_r, with_cnt)

        @pl.when(cid == 1)
        def _():
            run(xh1_r, False)

        plsc.subcore_barrier()

        @pl.when(sid < NS - 1)
        def _():
            pltpu.sync_copy(agg_sh.at[pl.ds(base, STRIPE)],
                            agg_out.at[cid, pl.ds(base, STRIPE)])

        @pl.when(sid == NS - 1)
        def _():
            pltpu.sync_copy(agg_sh.at[pl.ds((NS - 1) * STRIPE, STRIPE_LAST)],
                            agg_out.at[cid, pl.ds((NS - 1) * STRIPE,
                                                  STRIPE_LAST)])

        if with_cnt:
            @pl.when((cid == 0) & (sid < NS - 1))
            def _():
                pltpu.sync_copy(cnt_sh.at[pl.ds(base, STRIPE)],
                                cnt_out.at[pl.ds(base, STRIPE)])

            @pl.when((cid == 0) & (sid == NS - 1))
            def _():
                pltpu.sync_copy(
                    cnt_sh.at[pl.ds((NS - 1) * STRIPE, STRIPE_LAST)],
                    cnt_out.at[pl.ds((NS - 1) * STRIPE, STRIPE_LAST)])

    out_type = [jax.ShapeDtypeStruct((NC, N, HALF), jnp.float32)]
    scratch = [
        pltpu.VMEM_SHARED((TBL, HALF), jnp.float32),
        pltpu.VMEM((NCH, CH), jnp.int32),
        pltpu.VMEM((NCH, CH), jnp.int32),
        pltpu.VMEM((CH, HALF), jnp.float32),
        pltpu.VMEM((CH, HALF), jnp.float32),
        pltpu.SemaphoreType.DMA,
        pltpu.SemaphoreType.DMA,
    ]
    if with_cnt:
        out_type.append(jax.ShapeDtypeStruct((N, 8), jnp.float32))
        scratch += [
            pltpu.VMEM_SHARED((TBL, 8), jnp.float32),
            pltpu.VMEM((CH, 8), jnp.float32),
            pltpu.SemaphoreType.DMA,
        ]
    f = pl.kernel(body, out_type=out_type, mesh=mesh, scratch_types=scratch,
                  compiler_params=pltpu.CompilerParams(
                      use_tc_tiling_on_sc=False))
    return f(xh0, xh1, src3, dst3, zeros_h, zo16_h)


def _l1_body(agg3, cnt, x, wl, b, wr, hL_ref, hR_ref, stats_ref):
    i = pl.program_id(0)
    a = agg3[...]
    agg = jnp.concatenate([a[0], a[1]], axis=1)
    c = jnp.maximum(cnt[...][:, 0:1], 1.0)
    mean = agg / c
    o = (jnp.dot(mean, wl[...], preferred_element_type=jnp.float32)
         + jnp.dot(x[...], wr[...], preferred_element_type=jnp.float32)
         + b[...])
    nrm = jnp.sqrt(jnp.sum(o * o, axis=1, keepdims=True))
    o = o / jnp.maximum(nrm, 1e-12)
    h = jnp.maximum(o, 0.0)
    hL_ref[...] = h[:, :HALF]
    hR_ref[...] = h[:, HALF:]
    st = jnp.concatenate([jnp.sum(h, axis=0, keepdims=True),
                          jnp.sum(h * h, axis=0, keepdims=True)], axis=0)

    @pl.when(i == 0)
    def _():
        stats_ref[...] = st

    @pl.when(i > 0)
    def _():
        stats_ref[...] += st


def _layer1_dense(agg3, cnt, x, W_lT, b_l, W_rT):
    return pl.pallas_call(
        _l1_body,
        grid=(NBLK,),
        in_specs=[
            pl.BlockSpec((NC, ROWS, HALF), lambda i: (0, i, 0)),
            pl.BlockSpec((ROWS, 8), lambda i: (i, 0)),
            pl.BlockSpec((ROWS, D), lambda i: (i, 0)),
            pl.BlockSpec((D, D), lambda i: (0, 0)),
            pl.BlockSpec((1, D), lambda i: (0, 0)),
            pl.BlockSpec((D, D), lambda i: (0, 0)),
        ],
        out_specs=[
            pl.BlockSpec((ROWS, HALF), lambda i: (i, 0)),
            pl.BlockSpec((ROWS, HALF), lambda i: (i, 0)),
            pl.BlockSpec((2, D), lambda i: (0, 0)),
        ],
        out_shape=[
            jax.ShapeDtypeStruct((N, HALF), jnp.float32),
            jax.ShapeDtypeStruct((N, HALF), jnp.float32),
            jax.ShapeDtypeStruct((2, D), jnp.float32),
        ],
    )(agg3, cnt, x, W_lT, b_l, W_rT)


def _l2_body(agg3, cnt, hL, hR, wl, b, wr, scale, shift, out_ref):
    a = agg3[...]
    agg = jnp.concatenate([a[0], a[1]], axis=1)
    c0 = cnt[...][:, 0:1]
    mask = (c0 > 0.0).astype(jnp.float32)
    mean = (agg / jnp.maximum(c0, 1.0)) * scale[...] + mask * shift[...]
    h = jnp.concatenate([hL[...], hR[...]], axis=1) * scale[...] + shift[...]
    o = (jnp.dot(mean, wl[...], preferred_element_type=jnp.float32)
         + jnp.dot(h, wr[...], preferred_element_type=jnp.float32)
         + b[...])
    nrm = jnp.sqrt(jnp.sum(o * o, axis=1, keepdims=True))
    out_ref[...] = o / jnp.maximum(nrm, 1e-12)


def _layer2_dense(agg3, cnt, hL, hR, W_lT, b_l, W_rT, scale, shift):
    return pl.pallas_call(
        _l2_body,
        grid=(NBLK,),
        in_specs=[
            pl.BlockSpec((NC, ROWS, HALF), lambda i: (0, i, 0)),
            pl.BlockSpec((ROWS, 8), lambda i: (i, 0)),
            pl.BlockSpec((ROWS, HALF), lambda i: (i, 0)),
            pl.BlockSpec((ROWS, HALF), lambda i: (i, 0)),
            pl.BlockSpec((D, D), lambda i: (0, 0)),
            pl.BlockSpec((1, D), lambda i: (0, 0)),
            pl.BlockSpec((D, D), lambda i: (0, 0)),
            pl.BlockSpec((1, D), lambda i: (0, 0)),
            pl.BlockSpec((1, D), lambda i: (0, 0)),
        ],
        out_specs=pl.BlockSpec((ROWS, D), lambda i: (i, 0)),
        out_shape=jax.ShapeDtypeStruct((N, D), jnp.float32),
    )(agg3, cnt, hL, hR, W_lT, b_l, W_rT, scale, shift)


def kernel(x, edge_index, W1_l, b1_l, W1_r, gamma, beta, W2_l, b2_l, W2_r):
    src = edge_index[0].astype(jnp.int32)
    dst = edge_index[1].astype(jnp.int32)
    pad = E_PAD - E
    src3 = jnp.concatenate([src, jnp.zeros((pad,), jnp.int32)]
                           ).reshape(NS, NCH, CH)
    dst3 = jnp.concatenate([dst, jnp.full((pad,), N, jnp.int32)]
                           ).reshape(NS, NCH, CH)
    xL, xR = x[:, :HALF], x[:, HALF:]
    zeros_h = jnp.zeros((STRIPE, HALF), jnp.float32)
    zo8 = jnp.concatenate([jnp.zeros((STRIPE, 8), jnp.float32),
                           jnp.ones((CH, 8), jnp.float32)])

    agg1, cnt = _sc_agg(xL, xR, src3, dst3, zeros_h, zo8, with_cnt=True)
    hL, hR, stats = _layer1_dense(agg1, cnt, x, W1_l.T, b1_l[None, :],
                                  W1_r.T)
    mu = stats[0:1, :] / N
    var = stats[1:2, :] / N - mu * mu
    scale = gamma[None, :] * jax.lax.rsqrt(var + 1e-5)
    shift = beta[None, :] - mu * scale
    (agg2,) = _sc_agg(hL, hR, src3, dst3, zeros_h, zo8, with_cnt=False)
    out = _layer2_dense(agg2, cnt, hL, hR, W2_l.T, b2_l[None, :], W2_r.T,
                        scale, shift)
    return out


# speedup vs baseline: 1.0320x; 1.0320x over previous
import jax
import jax.numpy as jnp
from jax import lax
from jax.experimental import pallas as pl
from jax.experimental.pallas import tpu as pltpu
from jax.experimental.pallas import tpu_sc as plsc

N = 10000
E = 160000
D = 256
HALF = 128
ROWS = 1000
NBLK = N // ROWS

NC = 2
NS = 16
CH = 64
NCH = 158
EPT = NCH * CH
E_PAD = NS * EPT
STRIPE = 632
TBL = NS * STRIPE
STRIPE_LAST = N - (NS - 1) * STRIPE


def _sc_agg(xh0, xh1, src3, dst3, zeros_h, zo16_h, with_cnt):
    mesh = plsc.VectorSubcoreMesh(core_axis_name="c", subcore_axis_name="s")

    def body(*refs):
        if with_cnt:
            (xh0_r, xh1_r, src_h, dst_h, zr_h, zo16_r, agg_out, cnt_out,
             agg_sh, src_v, dst_v, rows_v, rows_b, sem, semb,
             cnt_sh, ones_v, semc) = refs
        else:
            (xh0_r, xh1_r, src_h, dst_h, zr_h, zo16_r, agg_out,
             agg_sh, src_v, dst_v, rows_v, rows_b, sem, semb) = refs
        cid = lax.axis_index("c")
        sid = lax.axis_index("s")
        base = sid * STRIPE

        pltpu.sync_copy(src_h.at[sid], src_v)
        pltpu.sync_copy(dst_h.at[sid], dst_v)

        @pl.when(cid == 0)
        def _():
            pltpu.async_copy(xh0_r.at[src_v.at[0]], rows_v, sem)

        @pl.when(cid == 1)
        def _():
            pltpu.async_copy(xh1_r.at[src_v.at[0]], rows_v, sem)
        pltpu.sync_copy(zr_h, agg_sh.at[pl.ds(base, STRIPE)])
        if with_cnt:
            @pl.when(cid == 0)
            def _():
                pltpu.sync_copy(zo16_r.at[pl.ds(0, STRIPE)],
                                cnt_sh.at[pl.ds(base, STRIPE)])
                pltpu.sync_copy(zo16_r.at[pl.ds(STRIPE, CH)], ones_v)

        def run(x_half, count):
            def scat(j, buf):
                if count:
                    ac = pltpu.async_copy(ones_v, cnt_sh.at[dst_v.at[j]],
                                          semc, add=True)
                pltpu.sync_copy(buf, agg_sh.at[dst_v.at[j]], add=True)
                if count:
                    ac.wait()

            def pair(k, carry):
                j0 = 2 * k
                g1 = pltpu.async_copy(x_half.at[src_v.at[j0 + 1]],
                                      rows_b, semb)
                pltpu.make_async_copy(x_half.at[src_v.at[j0]],
                                      rows_v, sem).wait()
                scat(j0, rows_v)

                @pl.when(j0 + 2 < NCH)
                def _():
                    pltpu.async_copy(x_half.at[src_v.at[j0 + 2]],
                                     rows_v, sem)
                g1.wait()
                scat(j0 + 1, rows_b)
                return carry
            lax.fori_loop(0, NCH // 2, pair, 0)

        plsc.subcore_barrier()

        @pl.when(cid == 0)
        def _():
            run(xh0_r, with_cnt)

        @pl.when(cid == 1)
        def _():
            run(xh1_r, False)

        plsc.subcore_barrier()

        @pl.when(sid < NS - 1)
        def _():
            pltpu.sync_copy(agg_sh.at[pl.ds(base, STRIPE)],
                            agg_out.at[cid, pl.ds(base, STRIPE)])

        @pl.when(sid == NS - 1)
        def _():
            pltpu.sync_copy(agg_sh.at[pl.ds((NS - 1) * STRIPE, STRIPE_LAST)],
                            agg_out.at[cid, pl.ds((NS - 1) * STRIPE,
                                                  STRIPE_LAST)])

        if with_cnt:
            @pl.when((cid == 0) & (sid < NS - 1))
            def _():
                pltpu.sync_copy(cnt_sh.at[pl.ds(base, STRIPE)],
                                cnt_out.at[pl.ds(base, STRIPE)])

            @pl.when((cid == 0) & (sid == NS - 1))
            def _():
                pltpu.sync_copy(
                    cnt_sh.at[pl.ds((NS - 1) * STRIPE, STRIPE_LAST)],
                    cnt_out.at[pl.ds((NS - 1) * STRIPE, STRIPE_LAST)])

    out_type = [jax.ShapeDtypeStruct((NC, N, HALF), jnp.float32)]
    scratch = [
        pltpu.VMEM_SHARED((TBL, HALF), jnp.float32),
        pltpu.VMEM((NCH, CH), jnp.int32),
        pltpu.VMEM((NCH, CH), jnp.int32),
        pltpu.VMEM((CH, HALF), jnp.float32),
        pltpu.VMEM((CH, HALF), jnp.float32),
        pltpu.SemaphoreType.DMA,
        pltpu.SemaphoreType.DMA,
    ]
    if with_cnt:
        out_type.append(jax.ShapeDtypeStruct((N, 8), jnp.float32))
        scratch += [
            pltpu.VMEM_SHARED((TBL, 8), jnp.float32),
            pltpu.VMEM((CH, 8), jnp.float32),
            pltpu.SemaphoreType.DMA,
        ]
    f = pl.kernel(body, out_type=out_type, mesh=mesh, scratch_types=scratch,
                  compiler_params=pltpu.CompilerParams(
                      use_tc_tiling_on_sc=False))
    return f(xh0, xh1, src3, dst3, zeros_h, zo16_h)


def _l1_body(agg3, cnt, x, wl, b, wr, hL_ref, hR_ref, stats_ref):
    i = pl.program_id(0)
    a = agg3[...]
    agg = jnp.concatenate([a[0], a[1]], axis=1)
    c = jnp.maximum(cnt[...][:, 0:1], 1.0)
    mean = agg / c
    o = (jnp.dot(mean, wl[...], preferred_element_type=jnp.float32)
         + jnp.dot(x[...], wr[...], preferred_element_type=jnp.float32)
         + b[...])
    nrm = jnp.sqrt(jnp.sum(o * o, axis=1, keepdims=True))
    o = o / jnp.maximum(nrm, 1e-12)
    h = jnp.maximum(o, 0.0)
    hL_ref[...] = h[:, :HALF]
    hR_ref[...] = h[:, HALF:]
    st = jnp.concatenate([jnp.sum(h, axis=0, keepdims=True),
                          jnp.sum(h * h, axis=0, keepdims=True)], axis=0)

    @pl.when(i == 0)
    def _():
        stats_ref[...] = st

    @pl.when(i > 0)
    def _():
        stats_ref[...] += st


def _layer1_dense(agg3, cnt, x, W_lT, b_l, W_rT):
    return pl.pallas_call(
        _l1_body,
        grid=(NBLK,),
        in_specs=[
            pl.BlockSpec((NC, ROWS, HALF), lambda i: (0, i, 0)),
            pl.BlockSpec((ROWS, 8), lambda i: (i, 0)),
            pl.BlockSpec((ROWS, D), lambda i: (i, 0)),
            pl.BlockSpec((D, D), lambda i: (0, 0)),
            pl.BlockSpec((1, D), lambda i: (0, 0)),
            pl.BlockSpec((D, D), lambda i: (0, 0)),
        ],
        out_specs=[
            pl.BlockSpec((ROWS, HALF), lambda i: (i, 0)),
            pl.BlockSpec((ROWS, HALF), lambda i: (i, 0)),
            pl.BlockSpec((2, D), lambda i: (0, 0)),
        ],
        out_shape=[
            jax.ShapeDtypeStruct((N, HALF), jnp.float32),
            jax.ShapeDtypeStruct((N, HALF), jnp.float32),
            jax.ShapeDtypeStruct((2, D), jnp.float32),
        ],
    )(agg3, cnt, x, W_lT, b_l, W_rT)


def _l2_body(agg3, cnt, hL, hR, wl, b, wr, offs, out_ref):
    a = agg3[...]
    agg = jnp.concatenate([a[0], a[1]], axis=1)
    c0 = cnt[...][:, 0:1]
    mask = (c0 > 0.0).astype(jnp.float32)
    mean = agg / jnp.maximum(c0, 1.0) + mask * offs[...]
    h = jnp.concatenate([hL[...], hR[...]], axis=1) + offs[...]
    o = (jnp.dot(mean, wl[...], preferred_element_type=jnp.float32)
         + jnp.dot(h, wr[...], preferred_element_type=jnp.float32)
         + b[...])
    nrm = jnp.sqrt(jnp.sum(o * o, axis=1, keepdims=True))
    out_ref[...] = o / jnp.maximum(nrm, 1e-12)


def _layer2_dense(agg3, cnt, hL, hR, W_lT, b_l, W_rT, offs):
    return pl.pallas_call(
        _l2_body,
        grid=(NBLK,),
        in_specs=[
            pl.BlockSpec((NC, ROWS, HALF), lambda i: (0, i, 0)),
            pl.BlockSpec((ROWS, 8), lambda i: (i, 0)),
            pl.BlockSpec((ROWS, HALF), lambda i: (i, 0)),
            pl.BlockSpec((ROWS, HALF), lambda i: (i, 0)),
            pl.BlockSpec((D, D), lambda i: (0, 0)),
            pl.BlockSpec((1, D), lambda i: (0, 0)),
            pl.BlockSpec((D, D), lambda i: (0, 0)),
            pl.BlockSpec((1, D), lambda i: (0, 0)),
        ],
        out_specs=pl.BlockSpec((ROWS, D), lambda i: (i, 0)),
        out_shape=jax.ShapeDtypeStruct((N, D), jnp.float32),
    )(agg3, cnt, hL, hR, W_lT, b_l, W_rT, offs)


def kernel(x, edge_index, W1_l, b1_l, W1_r, gamma, beta, W2_l, b2_l, W2_r):
    src = edge_index[0].astype(jnp.int32)
    dst = edge_index[1].astype(jnp.int32)
    pad = E_PAD - E
    src3 = jnp.concatenate([src, jnp.zeros((pad,), jnp.int32)]
                           ).reshape(NS, NCH, CH)
    dst3 = jnp.concatenate([dst, jnp.full((pad,), N, jnp.int32)]
                           ).reshape(NS, NCH, CH)
    xL, xR = x[:, :HALF], x[:, HALF:]
    zeros_h = jnp.zeros((STRIPE, HALF), jnp.float32)
    zo8 = jnp.concatenate([jnp.zeros((STRIPE, 8), jnp.float32),
                           jnp.ones((CH, 8), jnp.float32)])

    agg1, cnt = _sc_agg(xL, xR, src3, dst3, zeros_h, zo8, with_cnt=True)
    hL, hR, stats = _layer1_dense(agg1, cnt, x, W1_l.T, b1_l[None, :],
                                  W1_r.T)
    mu = stats[0:1, :] / N
    var = stats[1:2, :] / N - mu * mu
    scale = gamma[None, :] * jax.lax.rsqrt(var + 1e-5)
    offs = beta[None, :] / scale - mu
    W2lT = scale[0][:, None] * W2_l.T
    W2rT = scale[0][:, None] * W2_r.T
    (agg2,) = _sc_agg(hL, hR, src3, dst3, zeros_h, zo8, with_cnt=False)
    out = _layer2_dense(agg2, cnt, hL, hR, W2lT, b2_l[None, :], W2rT, offs)
    return out
